# baseline (speedup 1.0000x reference)
import functools

import jax
import jax.numpy as jnp
from jax.experimental import pallas as pl
from jax.experimental.pallas import tpu as pltpu


def _attn_body(scores_ref, mask_ref, valid_ref, reps_ref, out_ref, *, bt, rows):
    pid = pl.program_id(0)
    r0 = pid * bt

    s = scores_ref[pl.ds(r0, bt), :].astype(jnp.float32)
    keep = jnp.logical_and(mask_ref[pl.ds(r0, bt), :], valid_ref[pl.ds(r0, bt), :])
    s = jnp.where(keep, s, jnp.float32(-10000.0))
    mx = jnp.max(s, axis=-1, keepdims=True)
    e = jnp.exp(s - mx)
    att = e / jnp.sum(e, axis=-1, keepdims=True)

    for c0 in range(0, bt, rows):
        c1 = c0 + rows
        r = reps_ref[c0:c1, :, :].astype(jnp.float32)
        w = att[c0:c1, :]
        out = jnp.sum(w[:, :, None] * r, axis=1)
        out_ref[pl.ds(r0 + c0, rows), :] = out.astype(out_ref.dtype)


def _pick_bt(B, N, D, itemsize, target_bytes=3 << 20):
    row_bytes = max(1, N * D * itemsize)
    best = None
    for bt in range(8, B + 1, 8):
        if B % bt:
            continue
        if bt * row_bytes > target_bytes:
            break
        best = bt
    if best is not None:
        return best
    return min(B, 8)


def kernel(sentence_reps, sentence_mask, att_scores, valid_scores):
    B, N, D = sentence_reps.shape
    out_dtype = sentence_reps.dtype
    itemsize = sentence_reps.dtype.itemsize

    bt = _pick_bt(B, N, D, itemsize)
    grid = (pl.cdiv(B, bt),)

    rows = bt
    while rows > 8 and rows % 2 == 0 and rows * N * D * 4 > (3 << 19):
        rows //= 2

    reps_blk = bt * N * D * itemsize
    needed = 2 * reps_blk + (8 << 20)

    entry = pl.pallas_call(
        functools.partial(_attn_body, bt=bt, rows=rows),
        out_shape=jax.ShapeDtypeStruct((B, D), out_dtype),
        grid=grid,
        in_specs=[
            pl.BlockSpec((B, N), lambda b: (0, 0)),
            pl.BlockSpec((B, N), lambda b: (0, 0)),
            pl.BlockSpec((B, N), lambda b: (0, 0)),
            pl.BlockSpec((bt, N, D), lambda b: (b, 0, 0)),
        ],
        out_specs=pl.BlockSpec((B, D), lambda b: (0, 0)),
        compiler_params=pltpu.CompilerParams(
            dimension_semantics=("arbitrary",),
            vmem_limit_bytes=int(min(max(needed, 32 << 20), 58 << 20)),
        ),
    )
    return entry(att_scores, sentence_mask, valid_scores, sentence_reps)


# speedup vs baseline: 1.2791x; 1.2791x over previous
import functools

import jax
import jax.numpy as jnp
from jax.experimental import pallas as pl
from jax.experimental.pallas import tpu as pltpu


def _attn_body(scores_ref, mask_ref, valid_ref, reps_ref, out_ref, *, rows):
    bt, n = scores_ref.shape

    s = scores_ref[...].astype(jnp.float32)
    keep = jnp.logical_and(mask_ref[...], valid_ref[...])
    s = jnp.where(keep, s, jnp.float32(-10000.0))
    mx = jnp.max(s, axis=-1, keepdims=True)
    e = jnp.exp(s - mx)
    att = e / jnp.sum(e, axis=-1, keepdims=True)

    for c0 in range(0, bt, rows):
        c1 = c0 + rows
        r = reps_ref[c0:c1, :, :].astype(jnp.float32)
        w = att[c0:c1, :]
        out = jnp.sum(w[:, :, None] * r, axis=1)
        out_ref[c0:c1, :] = out.astype(out_ref.dtype)


def kernel(sentence_reps, sentence_mask, att_scores, valid_scores):
    B, N, D = sentence_reps.shape
    out_dtype = sentence_reps.dtype
    itemsize = sentence_reps.dtype.itemsize

    bt = 48
    grid = (pl.cdiv(B, bt),)

    rows = bt
    while rows > 8 and rows % 2 == 0 and rows * N * D * 4 > (3 << 19):
        rows //= 2

    reps_blk = bt * N * D * itemsize
    needed = 2 * reps_blk + (8 << 20)

    entry = pl.pallas_call(
        functools.partial(_attn_body, rows=rows),
        out_shape=jax.ShapeDtypeStruct((B, D), out_dtype),
        grid=grid,
        in_specs=[
            pl.BlockSpec((bt, N), lambda b: (b, 0)),
            pl.BlockSpec((bt, N), lambda b: (b, 0)),
            pl.BlockSpec((bt, N), lambda b: (b, 0)),
            pl.BlockSpec((bt, N, D), lambda b: (b, 0, 0)),
        ],
        out_specs=pl.BlockSpec((bt, D), lambda b: (b, 0)),
        compiler_params=pltpu.CompilerParams(
            dimension_semantics=("arbitrary",),
            vmem_limit_bytes=int(min(max(needed, 32 << 20), 58 << 20)),
        ),
    )
    return entry(att_scores, sentence_mask, valid_scores, sentence_reps)
